# baseline (speedup 1.0000x reference)
import functools

import numpy as np

import jax
import jax.numpy as jnp
from jax import lax
from jax.experimental import pallas as pl
from jax.experimental.pallas import tpu as pltpu
from jax.experimental.pallas import tpu_sc as plsc

D_MODEL = 64
NUM_WORKERS = 32
BCHUNK = 128
LANES = 16

@functools.lru_cache(maxsize=None)
def _make_gather(batch: int, hist: int):
    assert batch == NUM_WORKERS * BCHUNK and hist % 2 == 0
    n_btiles = batch // BCHUNK
    mesh = plsc.VectorSubcoreMesh(core_axis_name="c", subcore_axis_name="s")

    @functools.partial(
        pl.kernel,
        mesh=mesh,
        out_type=jax.ShapeDtypeStruct(
            (hist, D_MODEL // 8, n_btiles, 8, BCHUNK), jnp.float32
        ),
        scratch_types=[
            pltpu.VMEM((hist, BCHUNK), jnp.int32),
            pltpu.VMEM((BCHUNK, D_MODEL), jnp.float32),
            pltpu.VMEM((BCHUNK, D_MODEL), jnp.float32),
            pltpu.VMEM((D_MODEL, BCHUNK), jnp.float32),
            pltpu.VMEM((D_MODEL, BCHUNK), jnp.float32),
            pltpu.SemaphoreType.DMA,
            pltpu.SemaphoreType.DMA,
            pltpu.SemaphoreType.DMA,
            pltpu.SemaphoreType.DMA,
        ],
        compiler_params=pltpu.CompilerParams(
            use_tc_tiling_on_sc=False, needs_layout_passes=False
        ),
    )
    def gather_kernel(
        idx_hbm, table_hbm, out_hbm,
        idx_v, rows0, rows1, tile0, tile1,
        sem_g0, sem_g1, sem_o0, sem_o1,
    ):
        wid = lax.axis_index("s") * 2 + lax.axis_index("c")

        lane = lax.broadcasted_iota(jnp.int32, (LANES,), 0)
        cks = [lane + LANES * k for k in range(D_MODEL // LANES)]

        pltpu.sync_copy(idx_hbm.at[:, pl.ds(wid * BCHUNK, BCHUNK)], idx_v)

        def transpose(rows, tile):
            @plsc.parallel_loop(0, BCHUNK, unroll=16)
            def _(b):
                bs = jnp.full((LANES,), 0, jnp.int32) + b
                for k in range(D_MODEL // LANES):
                    v = rows[b, pl.ds(k * LANES, LANES)]
                    plsc.store_scatter(tile, [cks[k], bs], v)

        def gather(t, rows, sem):
            pltpu.async_copy(table_hbm.at[idx_v.at[t]], rows, sem)

        def gather_wait(t, rows, sem):
            pltpu.make_async_copy(table_hbm.at[idx_v.at[t]], rows, sem).wait()

        def put_out(t, tile, sem):
            for c8 in range(D_MODEL // 8):
                pltpu.async_copy(
                    tile.at[pl.ds(8 * c8, 8)], out_hbm.at[t, c8, wid], sem
                )

        def drain_out(tile, sem):
            for c8 in range(D_MODEL // 8):
                pltpu.make_async_copy(
                    out_hbm.at[0, c8, wid], tile.at[pl.ds(8 * c8, 8)], sem
                ).wait()

        gather(0, rows0, sem_g0)

        def body(i, carry):
            t0 = 2 * i
            t1 = t0 + 1
            gather(t1, rows1, sem_g1)
            gather_wait(t0, rows0, sem_g0)

            @pl.when(i > 0)
            def _():
                drain_out(tile0, sem_o0)

            transpose(rows0, tile0)
            put_out(t0, tile0, sem_o0)

            @pl.when(t0 + 2 < hist)
            def _():
                gather(t0 + 2, rows0, sem_g0)

            gather_wait(t1, rows1, sem_g1)

            @pl.when(i > 0)
            def _():
                drain_out(tile1, sem_o1)

            transpose(rows1, tile1)
            put_out(t1, tile1, sem_o1)
            return carry

        lax.fori_loop(0, hist // 2, body, 0)
        drain_out(tile0, sem_o0)
        drain_out(tile1, sem_o1)

    return gather_kernel


def kernel(x, table):
    b, h = x.shape
    idx_t = x.T.astype(jnp.int32)
    out5 = _make_gather(b, h)(idx_t, table)
    return out5.transpose(2, 4, 0, 1, 3).reshape(b, h, D_MODEL)


# speedup vs baseline: 1.0328x; 1.0328x over previous
import functools

import jax
import jax.numpy as jnp
from jax import lax
from jax.experimental import pallas as pl
from jax.experimental.pallas import tpu as pltpu
from jax.experimental.pallas import tpu_sc as plsc

D_MODEL = 64
NUM_WORKERS = 32
CHUNK = 512


@functools.lru_cache(maxsize=None)
def _make_gather(n_rows: int):
    assert n_rows % (NUM_WORKERS * CHUNK) == 0
    rows_per_worker = n_rows // NUM_WORKERS
    n_chunks = rows_per_worker // CHUNK
    mesh = plsc.VectorSubcoreMesh(core_axis_name="c", subcore_axis_name="s")

    @functools.partial(
        pl.kernel,
        mesh=mesh,
        out_type=jax.ShapeDtypeStruct((n_rows, D_MODEL), jnp.float32),
        scratch_types=[
            pltpu.VMEM((CHUNK,), jnp.int32),
            pltpu.VMEM((CHUNK, D_MODEL), jnp.float32),
            pltpu.SemaphoreType.DMA,
        ],
        compiler_params=pltpu.CompilerParams(use_tc_tiling_on_sc=False),
    )
    def gather_kernel(idx_hbm, table_hbm, out_hbm, idx_v, rows_v, sem):
        wid = lax.axis_index("s") * 2 + lax.axis_index("c")
        base = wid * rows_per_worker

        def body(i, carry):
            off = base + i * CHUNK
            pltpu.sync_copy(idx_hbm.at[pl.ds(off, CHUNK)], idx_v)
            pltpu.async_copy(table_hbm.at[idx_v], rows_v, sem).wait()
            pltpu.sync_copy(rows_v, out_hbm.at[pl.ds(off, CHUNK)])
            return carry

        lax.fori_loop(0, n_chunks, body, 0)

    return gather_kernel


def kernel(x, table):
    b, h = x.shape
    idx = x.reshape(-1).astype(jnp.int32)
    out = _make_gather(idx.shape[0])(idx, table)
    return out.reshape(b, h, D_MODEL)


# speedup vs baseline: 1.0660x; 1.0322x over previous
import functools

import jax
import jax.numpy as jnp
from jax import lax
from jax.experimental import pallas as pl
from jax.experimental.pallas import tpu as pltpu
from jax.experimental.pallas import tpu_sc as plsc

D_MODEL = 64
NUM_WORKERS = 32
CHUNK = 512


@functools.lru_cache(maxsize=None)
def _make_gather(n_rows: int):
    assert n_rows % (NUM_WORKERS * CHUNK) == 0
    rows_per_worker = n_rows // NUM_WORKERS
    n_chunks = rows_per_worker // CHUNK
    mesh = plsc.VectorSubcoreMesh(core_axis_name="c", subcore_axis_name="s")

    @functools.partial(
        pl.kernel,
        mesh=mesh,
        out_type=jax.ShapeDtypeStruct((n_rows, D_MODEL), jnp.float32),
        scratch_types=[
            pltpu.VMEM((CHUNK,), jnp.int32),
            pltpu.VMEM((CHUNK,), jnp.int32),
            pltpu.VMEM((CHUNK, D_MODEL), jnp.float32),
            pltpu.VMEM((CHUNK, D_MODEL), jnp.float32),
            pltpu.SemaphoreType.DMA,
            pltpu.SemaphoreType.DMA,
            pltpu.SemaphoreType.DMA,
            pltpu.SemaphoreType.DMA,
        ],
        compiler_params=pltpu.CompilerParams(use_tc_tiling_on_sc=False),
    )
    def gather_kernel(
        idx_hbm, table_hbm, out_hbm,
        idx0, idx1, rows0, rows1, sem_g0, sem_g1, sem_o0, sem_o1,
    ):
        wid = lax.axis_index("s") * 2 + lax.axis_index("c")
        base = wid * rows_per_worker

        def gather(i, idx_v, rows_v, sem):
            off = base + i * CHUNK
            pltpu.sync_copy(idx_hbm.at[pl.ds(off, CHUNK)], idx_v)
            pltpu.async_copy(table_hbm.at[idx_v], rows_v, sem)

        def gather_wait(idx_v, rows_v, sem):
            pltpu.make_async_copy(table_hbm.at[idx_v], rows_v, sem).wait()

        def put_out(i, rows_v, sem):
            off = base + i * CHUNK
            pltpu.async_copy(rows_v, out_hbm.at[pl.ds(off, CHUNK)], sem)

        def drain_out(rows_v, sem):
            pltpu.make_async_copy(
                out_hbm.at[pl.ds(base, CHUNK)], rows_v, sem
            ).wait()

        gather(0, idx0, rows0, sem_g0)

        def body(j, carry):
            i0 = 2 * j
            i1 = i0 + 1

            @pl.when(j > 0)
            def _():
                drain_out(rows1, sem_o1)

            gather(i1, idx1, rows1, sem_g1)
            gather_wait(idx0, rows0, sem_g0)
            put_out(i0, rows0, sem_o0)

            @pl.when(i0 + 2 < n_chunks)
            def _():
                drain_out(rows0, sem_o0)
                gather(i0 + 2, idx0, rows0, sem_g0)

            gather_wait(idx1, rows1, sem_g1)
            put_out(i1, rows1, sem_o1)
            return carry

        lax.fori_loop(0, n_chunks // 2, body, 0)
        drain_out(rows0, sem_o0)
        drain_out(rows1, sem_o1)

    return gather_kernel


def kernel(x, table):
    b, h = x.shape
    idx = x.reshape(-1).astype(jnp.int32)
    out = _make_gather(idx.shape[0])(idx, table)
    return out.reshape(b, h, D_MODEL)


# speedup vs baseline: 1.0670x; 1.0009x over previous
import functools

import jax
import jax.numpy as jnp
from jax import lax
from jax.experimental import pallas as pl
from jax.experimental.pallas import tpu as pltpu
from jax.experimental.pallas import tpu_sc as plsc

D_MODEL = 64
NUM_WORKERS = 32
CHUNK = 512


@functools.lru_cache(maxsize=None)
def _make_gather(n_rows: int):
    assert n_rows % (NUM_WORKERS * CHUNK * 2) == 0
    rows_per_worker = n_rows // NUM_WORKERS
    n_chunks = rows_per_worker // CHUNK
    mesh = plsc.VectorSubcoreMesh(core_axis_name="c", subcore_axis_name="s")

    @functools.partial(
        pl.kernel,
        mesh=mesh,
        out_type=jax.ShapeDtypeStruct((n_rows, D_MODEL), jnp.float32),
        scratch_types=[
            pltpu.VMEM((CHUNK,), jnp.int32),
            pltpu.VMEM((CHUNK,), jnp.int32),
            pltpu.VMEM((CHUNK, D_MODEL), jnp.float32),
            pltpu.VMEM((CHUNK, D_MODEL), jnp.float32),
            pltpu.SemaphoreType.DMA,
            pltpu.SemaphoreType.DMA,
            pltpu.SemaphoreType.DMA,
            pltpu.SemaphoreType.DMA,
        ],
        compiler_params=pltpu.CompilerParams(use_tc_tiling_on_sc=False),
    )
    def gather_kernel(
        idx_hbm, table_hbm, out_hbm,
        idx0, idx1, rows0, rows1, sem_g0, sem_g1, sem_o0, sem_o1,
    ):
        wid = lax.axis_index("s") * 2 + lax.axis_index("c")
        base = wid * rows_per_worker

        def gather(i, idx_v, rows_v, sem):
            off = base + i * CHUNK
            pltpu.sync_copy(idx_hbm.at[pl.ds(off, CHUNK)], idx_v)
            pltpu.async_copy(table_hbm.at[idx_v], rows_v, sem)

        def gather_wait(idx_v, rows_v, sem):
            pltpu.make_async_copy(table_hbm.at[idx_v], rows_v, sem).wait()

        def put_out(i, rows_v, sem):
            off = base + i * CHUNK
            pltpu.async_copy(rows_v, out_hbm.at[pl.ds(off, CHUNK)], sem)

        def drain_out(rows_v, sem):
            pltpu.make_async_copy(
                out_hbm.at[pl.ds(base, CHUNK)], rows_v, sem
            ).wait()

        gather(0, idx0, rows0, sem_g0)

        def body(j, carry):
            i0 = 2 * j
            i1 = i0 + 1

            @pl.when(j > 0)
            def _():
                drain_out(rows1, sem_o1)

            gather(i1, idx1, rows1, sem_g1)
            gather_wait(idx0, rows0, sem_g0)
            put_out(i0, rows0, sem_o0)

            @pl.when(i0 + 2 < n_chunks)
            def _():
                drain_out(rows0, sem_o0)
                gather(i0 + 2, idx0, rows0, sem_g0)

            gather_wait(idx1, rows1, sem_g1)
            put_out(i1, rows1, sem_o1)
            return carry

        lax.fori_loop(0, n_chunks // 2, body, 0)
        drain_out(rows0, sem_o0)
        drain_out(rows1, sem_o1)

    return gather_kernel


def kernel(x, table):
    b, h = x.shape
    idx = x.reshape(-1).astype(jnp.int32)
    out = _make_gather(idx.shape[0])(idx, table)
    return out.reshape(b, h, D_MODEL)
